# baseline (speedup 1.0000x reference)
import functools

import jax
import jax.numpy as jnp
from jax import lax
from jax.experimental import pallas as pl
from jax.experimental.pallas import tpu as pltpu
from jax.experimental.pallas import tpu_sc as plsc

N = 10000
E = 320000
D = 128
H0 = 64
H1 = 32
C = 47

NC = 2
NS = 16
NW = NC * NS
EPT = E // NW
CH = 80
NCHUNK = EPT // CH
NPAD = 10240
RPT = NPAD // NS
ZR = 80

NB = 5
BN = N // NB


def _scatter_body(with_deg, h, K, dt, p_hbm, edge_hbm, *refs):
    NGROUP = NCHUNK // K
    if with_deg:
        (out_hbm, deg_hbm, src_v, dst_v, rows, zbuf, acc, gsem, ssem,
         ones_v, zbufd, dacc, dsem) = refs
    else:
        out_hbm, src_v, dst_v, rows, zbuf, acc, gsem, ssem = refs
    cid = lax.axis_index("c")
    sid = lax.axis_index("s")
    wid = sid * NC + cid
    lanes = 16 if dt == jnp.float32 else 32

    def _zero_row(r, _):
        for cb in range(h // lanes):
            zbuf[r, pl.ds(cb * lanes, lanes)] = jnp.zeros((lanes,), dt)
        if with_deg:
            zbufd[r, pl.ds(0, 16)] = jnp.zeros((16,), jnp.float32)
            ones_v[r, pl.ds(0, 16)] = jnp.ones((16,), jnp.float32)
        return 0

    lax.fori_loop(0, ZR, _zero_row, 0)
    for i in range(RPT // ZR):
        pltpu.sync_copy(zbuf, acc.at[pl.ds(sid * RPT + i * ZR, ZR)])
        if with_deg:
            pltpu.sync_copy(zbufd, dacc.at[pl.ds(sid * RPT + i * ZR, ZR)])
    plsc.subcore_barrier()

    pltpu.sync_copy(edge_hbm.at[0, wid], src_v)
    pltpu.sync_copy(edge_hbm.at[1, wid], dst_v)

    def _fire_gathers(g, bank):
        for b in range(K):
            pltpu.async_copy(p_hbm.at[src_v.at[g * K + b]],
                             rows.at[bank * K + b], gsem)

    def _drain_gathers(g, bank):
        for b in range(K):
            pltpu.make_async_copy(p_hbm.at[src_v.at[g * K + b]],
                                  rows.at[bank * K + b], gsem).wait()

    def _fire_scatters(g, bank):
        for b in range(K):
            pltpu.async_copy(rows.at[bank * K + b],
                             acc.at[dst_v.at[g * K + b]], ssem, add=True)
            if with_deg:
                pltpu.async_copy(ones_v, dacc.at[dst_v.at[g * K + b]],
                                 dsem, add=True)

    def _drain_scatters(g, bank):
        for b in range(K):
            pltpu.make_async_copy(rows.at[bank * K + b],
                                  acc.at[dst_v.at[g * K + b]], ssem).wait()

    _fire_gathers(0, 0)

    def _group(g, _):
        bank = lax.rem(g, 2)
        _drain_gathers(g, bank)

        @pl.when(g > 0)
        def _():
            _drain_scatters(g - 1, 1 - bank)

        @pl.when(g < NGROUP - 1)
        def _():
            _fire_gathers(g + 1, 1 - bank)

        _fire_scatters(g, bank)
        return 0

    lax.fori_loop(0, NGROUP, _group, 0)
    _drain_scatters(NGROUP - 1, (NGROUP - 1) % 2)
    if with_deg:
        def _drain_deg(j, _):
            pltpu.make_async_copy(ones_v, dacc.at[dst_v.at[j]], dsem).wait()
            return 0
        lax.fori_loop(0, NCHUNK, _drain_deg, 0)
    plsc.subcore_barrier()

    rsl = pl.ds(sid * RPT, RPT)
    pltpu.sync_copy(acc.at[rsl], out_hbm.at[rsl, pl.ds(cid * h, h)])
    if with_deg:
        pltpu.sync_copy(dacc.at[rsl], deg_hbm.at[rsl, pl.ds(cid * 64, 16)])


def _make_scatter(h, with_deg, K, dt=jnp.float32):
    mesh = plsc.VectorSubcoreMesh(
        core_axis_name="c", subcore_axis_name="s", num_cores=NC, num_subcores=NS)
    out_type = [jax.ShapeDtypeStruct((NPAD, 128), dt)]
    scratch = [
        pltpu.VMEM((NCHUNK, CH), jnp.int32),
        pltpu.VMEM((NCHUNK, CH), jnp.int32),
        pltpu.VMEM((2 * K, CH, h), dt),
        pltpu.VMEM((ZR, h), dt),
        pltpu.VMEM_SHARED((NPAD, h), dt),
        pltpu.SemaphoreType.DMA,
        pltpu.SemaphoreType.DMA,
    ]
    if with_deg:
        out_type.append(jax.ShapeDtypeStruct((NPAD, 128), jnp.float32))
        scratch += [
            pltpu.VMEM((CH, 16), jnp.float32),
            pltpu.VMEM((ZR, 16), jnp.float32),
            pltpu.VMEM_SHARED((NPAD, 16), jnp.float32),
            pltpu.SemaphoreType.DMA,
        ]
    return pl.kernel(
        functools.partial(_scatter_body, with_deg, h, K, dt),
        out_type=out_type, mesh=mesh, scratch_types=scratch,
        compiler_params=pltpu.CompilerParams(use_tc_tiling_on_sc=False))


def _proj_body(x_ref, w_ref, o_ref):
    o_ref[...] = jnp.dot(x_ref[...], w_ref[...], preferred_element_type=jnp.float32)


def _inv_deg(degp_ref):
    deg = degp_ref[:, 0:1] + degp_ref[:, 64:65]
    return 1.0 / jnp.maximum(deg, 1.0)


def _layer0_body(feas_ref, part_ref, degp_ref, ws0_ref, wn1_ref,
                 b0_ref, h0_ref, p1_ref):
    inv = _inv_deg(degp_ref)
    agg = (part_ref[:, :H0] + part_ref[:, H0:]) * inv
    h0 = jnp.dot(feas_ref[...], ws0_ref[...], preferred_element_type=jnp.float32)
    h0 = jnp.maximum(h0 + agg + b0_ref[...], 0.0)
    h0_ref[...] = h0
    p1 = jnp.dot(h0, wn1_ref[...], preferred_element_type=jnp.float32)
    p1_ref[...] = p1.astype(jnp.bfloat16)


def _layer1_body(h0_ref, part_ref, degp_ref, ws1_ref, wlin_ref,
                 b1_ref, blin_ref, out_ref, h1_ref):
    inv = _inv_deg(degp_ref)
    part = part_ref[...].astype(jnp.float32)
    agg = (part[:, :H1] + part[:, H1:2 * H1]) * inv
    h1 = jnp.dot(h0_ref[...], ws1_ref[...], preferred_element_type=jnp.float32)
    h1 = h1 + agg + b1_ref[...]
    h1_ref[...] = h1
    out_ref[...] = jnp.dot(h1, wlin_ref[...], preferred_element_type=jnp.float32) + blin_ref[...]


def _rows(w):
    return pl.BlockSpec((BN, w), lambda i: (i, 0))


def _whole(shape):
    return pl.BlockSpec(shape, lambda i: (0,) * len(shape))


def kernel(feas, edge_index, W_self0, W_neigh0, b0, W_self1, W_neigh1, b1,
           W_lin, b_lin):
    edge_r = edge_index.astype(jnp.int32).reshape(2, NW, NCHUNK, CH)

    p0 = pl.pallas_call(
        _proj_body,
        grid=(NB,),
        in_specs=[_rows(D), _whole((D, H0))],
        out_specs=_rows(H0),
        out_shape=jax.ShapeDtypeStruct((N, H0), jnp.float32),
    )(feas, W_neigh0)

    part0, degp = _make_scatter(H0, True, 5)(p0, edge_r)

    h0, p1 = pl.pallas_call(
        _layer0_body,
        grid=(NB,),
        in_specs=[_rows(D), _rows(NC * H0), _rows(128), _whole((D, H0)),
                  _whole((H0, H1)), _whole((1, H0))],
        out_specs=[_rows(H0), _rows(H1)],
        out_shape=[jax.ShapeDtypeStruct((N, H0), jnp.float32),
                   jax.ShapeDtypeStruct((N, H1), jnp.bfloat16)],
    )(feas, part0, degp, W_self0, W_neigh1, b0.reshape(1, H0))

    (part1,) = _make_scatter(H1, False, 5, jnp.bfloat16)(p1, edge_r)

    out, h1 = pl.pallas_call(
        _layer1_body,
        grid=(NB,),
        in_specs=[_rows(H0), _rows(128), _rows(128), _whole((H0, H1)),
                  _whole((H1, C)), _whole((1, H1)), _whole((1, C))],
        out_specs=[_rows(C), _rows(H1)],
        out_shape=[jax.ShapeDtypeStruct((N, C), jnp.float32),
                   jax.ShapeDtypeStruct((N, H1), jnp.float32)],
    )(h0, part1, degp, W_self1, W_lin, b1.reshape(1, H1),
      b_lin.reshape(1, C))

    return (out, h1)


# speedup vs baseline: 1.0560x; 1.0560x over previous
import functools

import jax
import jax.numpy as jnp
from jax import lax
from jax.experimental import pallas as pl
from jax.experimental.pallas import tpu as pltpu
from jax.experimental.pallas import tpu_sc as plsc

N = 10000
E = 320000
D = 128
H0 = 64
H1 = 32
C = 47

NC = 2
NS = 16
NW = NC * NS
EPT = E // NW
CH = 80
NCHUNK = EPT // CH
NPAD = 10240
RPT = NPAD // NS
ZR = 80

NB = 5
BN = N // NB


def _scatter_body(with_deg, h, K, dt, p_hbm, edge_hbm, *refs):
    NGROUP = NCHUNK // K
    if with_deg:
        (out_hbm, deg_hbm, src_v, dst_v, rows, zbuf, acc, gsem, ssem,
         ones_v, zbufd, dacc, dsem) = refs
    else:
        out_hbm, src_v, dst_v, rows, zbuf, acc, gsem, ssem = refs
    cid = lax.axis_index("c")
    sid = lax.axis_index("s")
    wid = sid * NC + cid
    lanes = 16 if dt == jnp.float32 else 32

    def _zero_row(r, _):
        for cb in range(h // lanes):
            zbuf[r, pl.ds(cb * lanes, lanes)] = jnp.zeros((lanes,), dt)
        if with_deg:
            zbufd[r, pl.ds(0, 16)] = jnp.zeros((16,), jnp.float32)
            ones_v[r, pl.ds(0, 16)] = jnp.ones((16,), jnp.float32)
        return 0

    lax.fori_loop(0, ZR, _zero_row, 0)
    for i in range(RPT // ZR):
        pltpu.sync_copy(zbuf, acc.at[pl.ds(sid * RPT + i * ZR, ZR)])
        if with_deg:
            pltpu.sync_copy(zbufd, dacc.at[pl.ds(sid * RPT + i * ZR, ZR)])
    plsc.subcore_barrier()

    pltpu.sync_copy(edge_hbm.at[0, wid], src_v)
    pltpu.sync_copy(edge_hbm.at[1, wid], dst_v)

    def _fire_gathers(g, bank):
        for b in range(K):
            pltpu.async_copy(p_hbm.at[src_v.at[g * K + b]],
                             rows.at[bank * K + b], gsem)

    def _drain_gathers(g, bank):
        for b in range(K):
            pltpu.make_async_copy(p_hbm.at[src_v.at[g * K + b]],
                                  rows.at[bank * K + b], gsem).wait()

    def _fire_scatters(g, bank):
        for b in range(K):
            pltpu.async_copy(rows.at[bank * K + b],
                             acc.at[dst_v.at[g * K + b]], ssem, add=True)
            if with_deg:
                pltpu.async_copy(ones_v, dacc.at[dst_v.at[g * K + b]],
                                 dsem, add=True)

    def _drain_scatters(g, bank):
        for b in range(K):
            pltpu.make_async_copy(rows.at[bank * K + b],
                                  acc.at[dst_v.at[g * K + b]], ssem).wait()

    _fire_gathers(0, 0)

    def _group(g, _):
        bank = lax.rem(g, 2)
        _drain_gathers(g, bank)

        @pl.when(g > 0)
        def _():
            _drain_scatters(g - 1, 1 - bank)

        @pl.when(g < NGROUP - 1)
        def _():
            _fire_gathers(g + 1, 1 - bank)

        _fire_scatters(g, bank)
        return 0

    lax.fori_loop(0, NGROUP, _group, 0)
    _drain_scatters(NGROUP - 1, (NGROUP - 1) % 2)
    if with_deg:
        def _drain_deg(j, _):
            pltpu.make_async_copy(ones_v, dacc.at[dst_v.at[j]], dsem).wait()
            return 0
        lax.fori_loop(0, NCHUNK, _drain_deg, 0)
    plsc.subcore_barrier()

    rsl = pl.ds(sid * RPT, RPT)
    pltpu.sync_copy(acc.at[rsl], out_hbm.at[rsl, pl.ds(cid * h, h)])
    if with_deg:
        pltpu.sync_copy(dacc.at[rsl], deg_hbm.at[rsl, pl.ds(cid * 64, 16)])


def _make_scatter(h, with_deg, K, dt=jnp.float32):
    mesh = plsc.VectorSubcoreMesh(
        core_axis_name="c", subcore_axis_name="s", num_cores=NC, num_subcores=NS)
    out_type = [jax.ShapeDtypeStruct((NPAD, 128), dt)]
    scratch = [
        pltpu.VMEM((NCHUNK, CH), jnp.int32),
        pltpu.VMEM((NCHUNK, CH), jnp.int32),
        pltpu.VMEM((2 * K, CH, h), dt),
        pltpu.VMEM((ZR, h), dt),
        pltpu.VMEM_SHARED((NPAD, h), dt),
        pltpu.SemaphoreType.DMA,
        pltpu.SemaphoreType.DMA,
    ]
    if with_deg:
        out_type.append(jax.ShapeDtypeStruct((NPAD, 128), jnp.float32))
        scratch += [
            pltpu.VMEM((CH, 16), jnp.float32),
            pltpu.VMEM((ZR, 16), jnp.float32),
            pltpu.VMEM_SHARED((NPAD, 16), jnp.float32),
            pltpu.SemaphoreType.DMA,
        ]
    return pl.kernel(
        functools.partial(_scatter_body, with_deg, h, K, dt),
        out_type=out_type, mesh=mesh, scratch_types=scratch,
        compiler_params=pltpu.CompilerParams(use_tc_tiling_on_sc=False))


def _proj_body(x_ref, w_ref, o_ref):
    o_ref[...] = jnp.dot(x_ref[...], w_ref[...], preferred_element_type=jnp.float32)


def _inv_deg(degp_ref):
    deg = degp_ref[:, 0:1] + degp_ref[:, 64:65]
    return 1.0 / jnp.maximum(deg, 1.0)


def _layer0_body(feas_ref, part_ref, degp_ref, ws0_ref, wn1_ref,
                 b0_ref, h0_ref, p1_ref):
    inv = _inv_deg(degp_ref)
    agg = (part_ref[:, :H0] + part_ref[:, H0:]) * inv
    h0 = jnp.dot(feas_ref[...], ws0_ref[...], preferred_element_type=jnp.float32)
    h0 = jnp.maximum(h0 + agg + b0_ref[...], 0.0)
    h0_ref[...] = jnp.concatenate(
        [h0, inv, jnp.zeros((BN, 128 - H0 - 1), jnp.float32)], axis=1)
    p1_ref[...] = jnp.dot(h0, wn1_ref[...], preferred_element_type=jnp.float32)


def _layer1_body(h0_ref, part_ref, ws1_ref, wlin_ref,
                 b1_ref, blin_ref, out_ref, h1_ref):
    inv = h0_ref[:, H0:H0 + 1]
    agg = (part_ref[:, :H1] + part_ref[:, H1:2 * H1]) * inv
    h1 = jnp.dot(h0_ref[:, :H0], ws1_ref[...], preferred_element_type=jnp.float32)
    h1 = h1 + agg + b1_ref[...]
    h1_ref[...] = h1
    out_ref[...] = jnp.dot(h1, wlin_ref[...], preferred_element_type=jnp.float32) + blin_ref[...]


def _rows(w):
    return pl.BlockSpec((BN, w), lambda i: (i, 0))


def _whole(shape):
    return pl.BlockSpec(shape, lambda i: (0,) * len(shape))


def kernel(feas, edge_index, W_self0, W_neigh0, b0, W_self1, W_neigh1, b1,
           W_lin, b_lin):
    edge_r = edge_index.astype(jnp.int32).reshape(2, NW, NCHUNK, CH)

    p0 = pl.pallas_call(
        _proj_body,
        out_shape=jax.ShapeDtypeStruct((N, H0), jnp.float32),
    )(feas, W_neigh0)

    part0, degp = _make_scatter(H0, True, 5)(p0, edge_r)

    h0, p1 = pl.pallas_call(
        _layer0_body,
        grid=(NB,),
        in_specs=[_rows(D), _rows(NC * H0), _rows(128), _whole((D, H0)),
                  _whole((H0, H1)), _whole((1, H0))],
        out_specs=[_rows(128), _rows(H1)],
        out_shape=[jax.ShapeDtypeStruct((N, 128), jnp.float32),
                   jax.ShapeDtypeStruct((N, H1), jnp.float32)],
    )(feas, part0, degp, W_self0, W_neigh1, b0.reshape(1, H0))

    (part1,) = _make_scatter(H1, False, 5)(p1, edge_r)

    out, h1 = pl.pallas_call(
        _layer1_body,
        grid=(NB,),
        in_specs=[_rows(128), _rows(128), _whole((H0, H1)),
                  _whole((H1, C)), _whole((1, H1)), _whole((1, C))],
        out_specs=[_rows(C), _rows(H1)],
        out_shape=[jax.ShapeDtypeStruct((N, C), jnp.float32),
                   jax.ShapeDtypeStruct((N, H1), jnp.float32)],
    )(h0, part1, W_self1, W_lin, b1.reshape(1, H1),
      b_lin.reshape(1, C))

    return (out, h1)


# speedup vs baseline: 1.1174x; 1.0581x over previous
import functools

import jax
import jax.numpy as jnp
from jax import lax
from jax.experimental import pallas as pl
from jax.experimental.pallas import tpu as pltpu
from jax.experimental.pallas import tpu_sc as plsc

N = 10000
E = 320000
D = 128
H0 = 64
H1 = 32
C = 47

NC = 2
NS = 16
NW = NC * NS
EPT = E // NW
CH = 80
NCHUNK = EPT // CH
NPAD = 10240
RPT = NPAD // NS
ZR = 80

NB = 5
BN = N // NB


def _scatter_body(with_deg, h, K, dt, NBANK, p_hbm, edge_hbm, *refs):
    NGROUP = NCHUNK // K
    if with_deg:
        (out_hbm, deg_hbm, src_v, dst_v, rows, zbuf, acc, gsem, ssem,
         ones_v, zbufd, dacc, dsem) = refs
    else:
        out_hbm, src_v, dst_v, rows, zbuf, acc, gsem, ssem = refs
    cid = lax.axis_index("c")
    sid = lax.axis_index("s")
    wid = sid * NC + cid
    lanes = 16 if dt == jnp.float32 else 32

    def _zero_row(r, _):
        for cb in range(h // lanes):
            zbuf[r, pl.ds(cb * lanes, lanes)] = jnp.zeros((lanes,), dt)
        if with_deg:
            zbufd[r, pl.ds(0, 16)] = jnp.zeros((16,), jnp.float32)
            ones_v[r, pl.ds(0, 16)] = jnp.ones((16,), jnp.float32)
        return 0

    lax.fori_loop(0, ZR, _zero_row, 0)
    for i in range(RPT // ZR):
        pltpu.sync_copy(zbuf, acc.at[pl.ds(sid * RPT + i * ZR, ZR)])
        if with_deg:
            pltpu.sync_copy(zbufd, dacc.at[pl.ds(sid * RPT + i * ZR, ZR)])
    plsc.subcore_barrier()

    pltpu.sync_copy(edge_hbm.at[0, wid], src_v)
    pltpu.sync_copy(edge_hbm.at[1, wid], dst_v)

    def _fire_gathers(g, bank):
        for b in range(K):
            pltpu.async_copy(p_hbm.at[src_v.at[g * K + b]],
                             rows.at[bank * K + b], gsem)

    def _drain_gathers(g, bank):
        for b in range(K):
            pltpu.make_async_copy(p_hbm.at[src_v.at[g * K + b]],
                                  rows.at[bank * K + b], gsem).wait()

    def _fire_scatters(g, bank):
        for b in range(K):
            pltpu.async_copy(rows.at[bank * K + b],
                             acc.at[dst_v.at[g * K + b]], ssem, add=True)
            if with_deg:
                pltpu.async_copy(ones_v, dacc.at[dst_v.at[g * K + b]],
                                 dsem, add=True)

    def _drain_scatters(g, bank):
        for b in range(K):
            pltpu.make_async_copy(rows.at[bank * K + b],
                                  acc.at[dst_v.at[g * K + b]], ssem).wait()

    for i in range(NBANK - 1):
        _fire_gathers(i, i)

    def _group(g, _):
        bank = lax.rem(g, NBANK)
        _drain_gathers(g, bank)

        @pl.when(g > 0)
        def _():
            _drain_scatters(g - 1, lax.rem(g - 1, NBANK))

        @pl.when(g + NBANK - 1 < NGROUP)
        def _():
            gn = g + NBANK - 1
            _fire_gathers(gn, lax.rem(gn, NBANK))

        _fire_scatters(g, bank)
        return 0

    lax.fori_loop(0, NGROUP, _group, 0)
    _drain_scatters(NGROUP - 1, (NGROUP - 1) % NBANK)
    if with_deg:
        def _drain_deg(j, _):
            pltpu.make_async_copy(ones_v, dacc.at[dst_v.at[j]], dsem).wait()
            return 0
        lax.fori_loop(0, NCHUNK, _drain_deg, 0)
    plsc.subcore_barrier()

    rsl = pl.ds(sid * RPT, RPT)
    pltpu.sync_copy(acc.at[rsl], out_hbm.at[rsl, pl.ds(cid * h, h)])
    if with_deg:
        pltpu.sync_copy(dacc.at[rsl], deg_hbm.at[rsl, pl.ds(cid * 64, 16)])


def _make_scatter(h, with_deg, K, dt=jnp.float32, NBANK=2):
    mesh = plsc.VectorSubcoreMesh(
        core_axis_name="c", subcore_axis_name="s", num_cores=NC, num_subcores=NS)
    out_type = [jax.ShapeDtypeStruct((NPAD, 128), dt)]
    scratch = [
        pltpu.VMEM((NCHUNK, CH), jnp.int32),
        pltpu.VMEM((NCHUNK, CH), jnp.int32),
        pltpu.VMEM((NBANK * K, CH, h), dt),
        pltpu.VMEM((ZR, h), dt),
        pltpu.VMEM_SHARED((NPAD, h), dt),
        pltpu.SemaphoreType.DMA,
        pltpu.SemaphoreType.DMA,
    ]
    if with_deg:
        out_type.append(jax.ShapeDtypeStruct((NPAD, 128), jnp.float32))
        scratch += [
            pltpu.VMEM((CH, 16), jnp.float32),
            pltpu.VMEM((ZR, 16), jnp.float32),
            pltpu.VMEM_SHARED((NPAD, 16), jnp.float32),
            pltpu.SemaphoreType.DMA,
        ]
    return pl.kernel(
        functools.partial(_scatter_body, with_deg, h, K, dt, NBANK),
        out_type=out_type, mesh=mesh, scratch_types=scratch,
        compiler_params=pltpu.CompilerParams(use_tc_tiling_on_sc=False))


def _proj_body(x_ref, w_ref, o_ref):
    o_ref[...] = jnp.dot(x_ref[...], w_ref[...], preferred_element_type=jnp.float32)


def _inv_deg(degp_ref):
    deg = degp_ref[:, 0:1] + degp_ref[:, 64:65]
    return 1.0 / jnp.maximum(deg, 1.0)


def _layer0_body(feas_ref, part_ref, degp_ref, ws0_ref, wn1_ref,
                 b0_ref, h0_ref, p1_ref):
    inv = _inv_deg(degp_ref)
    agg = (part_ref[:, :H0] + part_ref[:, H0:]) * inv
    h0 = jnp.dot(feas_ref[...], ws0_ref[...], preferred_element_type=jnp.float32)
    h0 = jnp.maximum(h0 + agg + b0_ref[...], 0.0)
    h0_ref[...] = jnp.concatenate(
        [h0, inv, jnp.zeros((BN, 128 - H0 - 1), jnp.float32)], axis=1)
    p1_ref[...] = jnp.dot(h0, wn1_ref[...], preferred_element_type=jnp.float32)


def _layer1_body(h0_ref, part_ref, ws1_ref, wlin_ref,
                 b1_ref, blin_ref, out_ref, h1_ref):
    inv = h0_ref[:, H0:H0 + 1]
    agg = (part_ref[:, :H1] + part_ref[:, H1:2 * H1]) * inv
    h1 = jnp.dot(h0_ref[:, :H0], ws1_ref[...], preferred_element_type=jnp.float32)
    h1 = h1 + agg + b1_ref[...]
    h1_ref[...] = h1
    out_ref[...] = jnp.dot(h1, wlin_ref[...], preferred_element_type=jnp.float32) + blin_ref[...]


def _rows(w):
    return pl.BlockSpec((BN, w), lambda i: (i, 0))


def _whole(shape):
    return pl.BlockSpec(shape, lambda i: (0,) * len(shape))


def kernel(feas, edge_index, W_self0, W_neigh0, b0, W_self1, W_neigh1, b1,
           W_lin, b_lin):
    edge_r = edge_index.astype(jnp.int32).reshape(2, NW, NCHUNK, CH)

    p0 = pl.pallas_call(
        _proj_body,
        out_shape=jax.ShapeDtypeStruct((N, H0), jnp.float32),
    )(feas, W_neigh0)

    part0, degp = _make_scatter(H0, True, 5)(p0, edge_r)

    h0, p1 = pl.pallas_call(
        _layer0_body,
        grid=(NB,),
        in_specs=[_rows(D), _rows(NC * H0), _rows(128), _whole((D, H0)),
                  _whole((H0, H1)), _whole((1, H0))],
        out_specs=[_rows(128), _rows(H1)],
        out_shape=[jax.ShapeDtypeStruct((N, 128), jnp.float32),
                   jax.ShapeDtypeStruct((N, H1), jnp.float32)],
    )(feas, part0, degp, W_self0, W_neigh1, b0.reshape(1, H0))

    (part1,) = _make_scatter(H1, False, 5, NBANK=3)(p1, edge_r)

    out, h1 = pl.pallas_call(
        _layer1_body,
        grid=(NB,),
        in_specs=[_rows(128), _rows(128), _whole((H0, H1)),
                  _whole((H1, C)), _whole((1, H1)), _whole((1, C))],
        out_specs=[_rows(C), _rows(H1)],
        out_shape=[jax.ShapeDtypeStruct((N, C), jnp.float32),
                   jax.ShapeDtypeStruct((N, H1), jnp.float32)],
    )(h0, part1, W_self1, W_lin, b1.reshape(1, H1),
      b_lin.reshape(1, C))

    return (out, h1)


# speedup vs baseline: 1.1200x; 1.0023x over previous
import functools

import jax
import jax.numpy as jnp
from jax import lax
from jax.experimental import pallas as pl
from jax.experimental.pallas import tpu as pltpu
from jax.experimental.pallas import tpu_sc as plsc

N = 10000
E = 320000
D = 128
H0 = 64
H1 = 32
C = 47

NC = 2
NS = 16
NW = NC * NS
EPT = E // NW
CH = 80
NCHUNK = EPT // CH
NPAD = 10240
RPT = NPAD // NS
ZR = 80

NB = 5
BN = N // NB


def _scatter_body(with_deg, h, K, dt, NBANK, p_hbm, edge_hbm, *refs):
    NGROUP = NCHUNK // K
    if with_deg:
        (out_hbm, deg_hbm, src_v, dst_v, rows, zbuf, acc, gsem, ssem,
         ones_v, zbufd, dacc, dsem) = refs
    else:
        out_hbm, src_v, dst_v, rows, zbuf, acc, gsem, ssem = refs
    cid = lax.axis_index("c")
    sid = lax.axis_index("s")
    wid = sid * NC + cid
    lanes = 16 if dt == jnp.float32 else 32

    def _zero_row(r, _):
        for cb in range(h // lanes):
            zbuf[r, pl.ds(cb * lanes, lanes)] = jnp.zeros((lanes,), dt)
        if with_deg:
            zbufd[r, pl.ds(0, 16)] = jnp.zeros((16,), jnp.float32)
            ones_v[r, pl.ds(0, 16)] = jnp.ones((16,), jnp.float32)
        return 0

    lax.fori_loop(0, ZR, _zero_row, 0)
    for i in range(RPT // ZR):
        pltpu.sync_copy(zbuf, acc.at[pl.ds(sid * RPT + i * ZR, ZR)])
        if with_deg:
            pltpu.sync_copy(zbufd, dacc.at[pl.ds(sid * RPT + i * ZR, ZR)])
    plsc.subcore_barrier()

    pltpu.sync_copy(edge_hbm.at[0, wid], src_v)
    pltpu.sync_copy(edge_hbm.at[1, wid], dst_v)

    def _fire_gathers(g, bank):
        for b in range(K):
            pltpu.async_copy(p_hbm.at[src_v.at[g * K + b]],
                             rows.at[bank * K + b], gsem)

    def _drain_gathers(g, bank):
        for b in range(K):
            pltpu.make_async_copy(p_hbm.at[src_v.at[g * K + b]],
                                  rows.at[bank * K + b], gsem).wait()

    def _fire_scatters(g, bank):
        for b in range(K):
            pltpu.async_copy(rows.at[bank * K + b],
                             acc.at[dst_v.at[g * K + b]], ssem, add=True)
            if with_deg:
                pltpu.async_copy(ones_v, dacc.at[dst_v.at[g * K + b]],
                                 dsem, add=True)

    def _drain_scatters(g, bank):
        for b in range(K):
            pltpu.make_async_copy(rows.at[bank * K + b],
                                  acc.at[dst_v.at[g * K + b]], ssem).wait()

    for i in range(NBANK - 1):
        _fire_gathers(i, i)

    def _group(g, _):
        bank = lax.rem(g, NBANK)
        _drain_gathers(g, bank)

        @pl.when(g > 0)
        def _():
            _drain_scatters(g - 1, lax.rem(g - 1, NBANK))

        @pl.when(g + NBANK - 1 < NGROUP)
        def _():
            gn = g + NBANK - 1
            _fire_gathers(gn, lax.rem(gn, NBANK))

        _fire_scatters(g, bank)
        return 0

    lax.fori_loop(0, NGROUP, _group, 0)
    _drain_scatters(NGROUP - 1, (NGROUP - 1) % NBANK)
    if with_deg:
        def _drain_deg(j, _):
            pltpu.make_async_copy(ones_v, dacc.at[dst_v.at[j]], dsem).wait()
            return 0
        lax.fori_loop(0, NCHUNK, _drain_deg, 0)
    plsc.subcore_barrier()

    rsl = pl.ds(sid * RPT, RPT)
    pltpu.sync_copy(acc.at[rsl], out_hbm.at[rsl, pl.ds(cid * h, h)])
    if with_deg:
        pltpu.sync_copy(dacc.at[rsl], deg_hbm.at[rsl, pl.ds(cid * 64, 16)])


def _make_scatter(h, with_deg, K, dt=jnp.float32, NBANK=2):
    mesh = plsc.VectorSubcoreMesh(
        core_axis_name="c", subcore_axis_name="s", num_cores=NC, num_subcores=NS)
    out_type = [jax.ShapeDtypeStruct((NPAD, 128), dt)]
    scratch = [
        pltpu.VMEM((NCHUNK, CH), jnp.int32),
        pltpu.VMEM((NCHUNK, CH), jnp.int32),
        pltpu.VMEM((NBANK * K, CH, h), dt),
        pltpu.VMEM((ZR, h), dt),
        pltpu.VMEM_SHARED((NPAD, h), dt),
        pltpu.SemaphoreType.DMA,
        pltpu.SemaphoreType.DMA,
    ]
    if with_deg:
        out_type.append(jax.ShapeDtypeStruct((NPAD, 128), jnp.float32))
        scratch += [
            pltpu.VMEM((CH, 16), jnp.float32),
            pltpu.VMEM((ZR, 16), jnp.float32),
            pltpu.VMEM_SHARED((NPAD, 16), jnp.float32),
            pltpu.SemaphoreType.DMA,
        ]
    return pl.kernel(
        functools.partial(_scatter_body, with_deg, h, K, dt, NBANK),
        out_type=out_type, mesh=mesh, scratch_types=scratch,
        compiler_params=pltpu.CompilerParams(use_tc_tiling_on_sc=False))


def _proj_body(x_ref, w_ref, o_ref):
    o_ref[...] = jnp.dot(x_ref[...], w_ref[...], preferred_element_type=jnp.float32)


def _inv_deg(degp_ref):
    deg = degp_ref[:, 0:1] + degp_ref[:, 64:65]
    return 1.0 / jnp.maximum(deg, 1.0)


def _layer0_body(feas_ref, part_ref, degp_ref, ws0_ref, wn1_ref,
                 b0_ref, h0_ref, p1_ref):
    inv = _inv_deg(degp_ref)
    agg = (part_ref[:, :H0] + part_ref[:, H0:]) * inv
    h0 = jnp.dot(feas_ref[...], ws0_ref[...], preferred_element_type=jnp.float32)
    h0 = jnp.maximum(h0 + agg + b0_ref[...], 0.0)
    h0_ref[...] = jnp.concatenate(
        [h0, inv, jnp.zeros((BN, 128 - H0 - 1), jnp.float32)], axis=1)
    p1_ref[...] = jnp.dot(h0, wn1_ref[...], preferred_element_type=jnp.float32)


def _layer1_body(h0_ref, part_ref, ws1_ref, wlin_ref,
                 b1_ref, blin_ref, out_ref, h1_ref):
    inv = h0_ref[:, H0:H0 + 1]
    agg = (part_ref[:, :H1] + part_ref[:, H1:2 * H1]) * inv
    h1 = jnp.dot(h0_ref[:, :H0], ws1_ref[...], preferred_element_type=jnp.float32)
    h1 = h1 + agg + b1_ref[...]
    h1_ref[...] = h1
    out_ref[...] = jnp.dot(h1, wlin_ref[...], preferred_element_type=jnp.float32) + blin_ref[...]


def _rows(w):
    return pl.BlockSpec((BN, w), lambda i: (i, 0))


def _whole(shape):
    return pl.BlockSpec(shape, lambda i: (0,) * len(shape))


def kernel(feas, edge_index, W_self0, W_neigh0, b0, W_self1, W_neigh1, b1,
           W_lin, b_lin):
    edge_r = edge_index.astype(jnp.int32).reshape(2, NW, NCHUNK, CH)

    p0 = pl.pallas_call(
        _proj_body,
        out_shape=jax.ShapeDtypeStruct((N, H0), jnp.float32),
    )(feas, W_neigh0)

    part0, degp = _make_scatter(H0, True, 5)(p0, edge_r)

    h0, p1 = pl.pallas_call(
        _layer0_body,
        grid=(NB,),
        in_specs=[_rows(D), _rows(NC * H0), _rows(128), _whole((D, H0)),
                  _whole((H0, H1)), _whole((1, H0))],
        out_specs=[_rows(128), _rows(H1)],
        out_shape=[jax.ShapeDtypeStruct((N, 128), jnp.float32),
                   jax.ShapeDtypeStruct((N, H1), jnp.float32)],
    )(feas, part0, degp, W_self0, W_neigh1, b0.reshape(1, H0))

    (part1,) = _make_scatter(H1, False, 5, NBANK=4)(p1, edge_r)

    out, h1 = pl.pallas_call(
        _layer1_body,
        grid=(NB,),
        in_specs=[_rows(128), _rows(128), _whole((H0, H1)),
                  _whole((H1, C)), _whole((1, H1)), _whole((1, C))],
        out_specs=[_rows(C), _rows(H1)],
        out_shape=[jax.ShapeDtypeStruct((N, C), jnp.float32),
                   jax.ShapeDtypeStruct((N, H1), jnp.float32)],
    )(h0, part1, W_self1, W_lin, b1.reshape(1, H1),
      b_lin.reshape(1, C))

    return (out, h1)
